# baseline (speedup 1.0000x reference)
import functools

import numpy as np
import jax
import jax.numpy as jnp
from jax import lax
from jax.experimental import pallas as pl
from jax.experimental.pallas import tpu as pltpu
from jax.experimental.pallas import tpu_sc as plsc

_N = 30
_NC2 = _N * (_N - 1) // 2
_PAD = 448
_NCHUNK = _PAD // 16
_TAIL = _NC2 - (_NCHUNK - 1) * 16


def _build_pair_rows():
    ra = np.zeros((_PAD,), dtype=np.int32)
    rb = np.zeros((_PAD,), dtype=np.int32)
    p = 0
    for i2 in range(_N):
        for j2 in range(i2):
            ra[p] = i2
            rb[p] = j2
            p += 1
    return ra, rb


_ROW_A, _ROW_B = _build_pair_rows()


def _sc_body(nrf_hbm, ra_hbm, rb_hbm, zeros_hbm, out_hbm,
             nrf_v, ra_v, rb_v, plane):
    num_cores = lax.axis_size("c")
    num_sub = lax.axis_size("s")
    wid = lax.axis_index("s") * num_cores + lax.axis_index("c")
    nw = num_cores * num_sub
    batch = nrf_hbm.shape[0]
    rows = batch // nw
    base = wid * rows

    pltpu.sync_copy(ra_hbm, ra_v)
    pltpu.sync_copy(rb_hbm, rb_v)
    pltpu.sync_copy(zeros_hbm, plane)

    def row_body(r, carry):
        b = base + r
        pltpu.sync_copy(nrf_hbm.at[b], nrf_v.at[pl.ds(0, _NC2)])
        lane = lax.iota(jnp.int32, 16)
        for j in range(_NCHUNK):
            sl = pl.ds(j * 16, 16)
            v = nrf_v[sl] * 0.5
            col = lane + (j * 16)
            ra = ra_v[sl]
            rb = rb_v[sl]
            if j == _NCHUNK - 1:
                mask = lane < _TAIL
                plsc.store_scatter(plane, [ra, col], v, mask=mask)
                plsc.store_scatter(plane, [rb, col], v, mask=mask)
            else:
                plsc.store_scatter(plane, [ra, col], v)
                plsc.store_scatter(plane, [rb, col], v)
        pltpu.sync_copy(plane, out_hbm.at[b])
        return carry

    lax.fori_loop(0, rows, row_body, 0)


def kernel(_NRF):
    b = _NRF.shape[0]
    mesh = plsc.VectorSubcoreMesh(core_axis_name="c", subcore_axis_name="s")
    sc_call = pl.kernel(
        _sc_body,
        out_type=jax.ShapeDtypeStruct((b, _N, _NC2), _NRF.dtype),
        mesh=mesh,
        scratch_types=[
            pltpu.VMEM((_PAD,), jnp.float32),
            pltpu.VMEM((_PAD,), jnp.int32),
            pltpu.VMEM((_PAD,), jnp.int32),
            pltpu.VMEM((_N, _NC2), jnp.float32),
        ],
        compiler_params=pltpu.CompilerParams(
            use_tc_tiling_on_sc=False, needs_layout_passes=False),
    )
    return sc_call(
        _NRF,
        jnp.asarray(_ROW_A),
        jnp.asarray(_ROW_B),
        jnp.zeros((_N, _NC2), jnp.float32),
    )


# speedup vs baseline: 1.1833x; 1.1833x over previous
import numpy as np
import jax
import jax.numpy as jnp
from jax import lax
from jax.experimental import pallas as pl
from jax.experimental.pallas import tpu as pltpu
from jax.experimental.pallas import tpu_sc as plsc

_N = 30
_NC2 = _N * (_N - 1) // 2
_PAD = 448
_NCHUNK = _PAD // 16
_TAIL = _NC2 - (_NCHUNK - 1) * 16


def _build_pair_rows():
    ra = np.zeros((_PAD,), dtype=np.int32)
    rb = np.zeros((_PAD,), dtype=np.int32)
    p = 0
    for i2 in range(_N):
        for j2 in range(i2):
            ra[p] = i2
            rb[p] = j2
            p += 1
    return ra, rb


_ROW_A, _ROW_B = _build_pair_rows()


def _scatter_row(nrf_v, ra_v, rb_v, plane):
    lane = lax.iota(jnp.int32, 16)
    for j in range(_NCHUNK):
        sl = pl.ds(j * 16, 16)
        v = nrf_v[sl] * 0.5
        col = lane + (j * 16)
        ra = ra_v[sl]
        rb = rb_v[sl]
        if j == _NCHUNK - 1:
            mask = lane < _TAIL
            plsc.store_scatter(plane, [ra, col], v, mask=mask)
            plsc.store_scatter(plane, [rb, col], v, mask=mask)
        else:
            plsc.store_scatter(plane, [ra, col], v)
            plsc.store_scatter(plane, [rb, col], v)


def _sc_body(nrf_hbm, ra_hbm, rb_hbm, zeros_hbm, out_hbm,
             nrf0, nrf1, ra_v, rb_v, p0, p1, sem_in0, sem_in1,
             sem_out0, sem_out1):
    num_cores = lax.axis_size("c")
    num_sub = lax.axis_size("s")
    wid = lax.axis_index("s") * num_cores + lax.axis_index("c")
    nw = num_cores * num_sub
    batch = nrf_hbm.shape[0]
    rows = batch // nw
    base = wid * rows
    planes = (p0, p1)
    nrfs = (nrf0, nrf1)
    sems_in = (sem_in0, sem_in1)
    sems_out = (sem_out0, sem_out1)

    pltpu.sync_copy(ra_hbm, ra_v)
    pltpu.sync_copy(rb_hbm, rb_v)
    pltpu.sync_copy(zeros_hbm, p0)
    pltpu.sync_copy(zeros_hbm, p1)

    pltpu.async_copy(nrf_hbm.at[base], nrf0.at[pl.ds(0, _NC2)], sem_in0)
    pltpu.async_copy(nrf_hbm.at[base + 1], nrf1.at[pl.ds(0, _NC2)], sem_in1)

    def step(rr, carry):
        for q in (0, 1):
            r = rr * 2 + q
            b = base + r
            @pl.when(rr >= 1)
            def _():
                pltpu.make_async_copy(planes[q], out_hbm.at[b],
                                      sems_out[q]).wait()
            pltpu.make_async_copy(nrf_hbm.at[b],
                                  nrfs[q].at[pl.ds(0, _NC2)],
                                  sems_in[q]).wait()
            _scatter_row(nrfs[q], ra_v, rb_v, planes[q])
            @pl.when(rr < (rows // 2) - 1)
            def _():
                pltpu.async_copy(nrf_hbm.at[b + 2],
                                 nrfs[q].at[pl.ds(0, _NC2)], sems_in[q])
            pltpu.async_copy(planes[q], out_hbm.at[b], sems_out[q])
        return carry

    lax.fori_loop(0, rows // 2, step, 0)

    pltpu.make_async_copy(p0, out_hbm.at[base + rows - 2], sem_out0).wait()
    pltpu.make_async_copy(p1, out_hbm.at[base + rows - 1], sem_out1).wait()


def kernel(_NRF):
    b = _NRF.shape[0]
    mesh = plsc.VectorSubcoreMesh(core_axis_name="c", subcore_axis_name="s")
    sc_call = pl.kernel(
        _sc_body,
        out_type=jax.ShapeDtypeStruct((b, _N, _NC2), _NRF.dtype),
        mesh=mesh,
        scratch_types=[
            pltpu.VMEM((_PAD,), jnp.float32),
            pltpu.VMEM((_PAD,), jnp.float32),
            pltpu.VMEM((_PAD,), jnp.int32),
            pltpu.VMEM((_PAD,), jnp.int32),
            pltpu.VMEM((_N, _NC2), jnp.float32),
            pltpu.VMEM((_N, _NC2), jnp.float32),
            pltpu.SemaphoreType.DMA,
            pltpu.SemaphoreType.DMA,
            pltpu.SemaphoreType.DMA,
            pltpu.SemaphoreType.DMA,
        ],
        compiler_params=pltpu.CompilerParams(
            use_tc_tiling_on_sc=False, needs_layout_passes=False),
    )
    return sc_call(
        _NRF,
        jnp.asarray(_ROW_A),
        jnp.asarray(_ROW_B),
        jnp.zeros((_N, _NC2), jnp.float32),
    )
